# baseline (speedup 1.0000x reference)
import functools

import jax
import jax.numpy as jnp
from jax import lax
from jax.experimental import pallas as pl
from jax.experimental.pallas import tpu as pltpu
from jax.experimental.pallas import tpu_sc as plsc

D_MODEL_K = 64
SCALE_K = 8.0

_NC = 2
_NS = 16
_NW = _NC * _NS
_LANES = 16

_V = 1000000
_NPAIR = _V // 2
_NWIN = _V // 128
_TAILP = (_NWIN * 128) // 2

_J = 200
_I = 4096
_P = 128
_NG = 4
_NO = 2


def _fmt_kernel(tT_hbm, tail_hbm, tpk_hbm, fb0, fb1, gb0, gb1, tb,
                sr0, sr1, sw0, sw1, st):
    fbufs = (fb0, fb1)
    gbufs = (gb0, gb1)
    srs = (sr0, sr1)
    sws = (sw0, sw1)
    wid = lax.axis_index("s") * _NC + lax.axis_index("c")

    kv = [
        lax.bitwise_and(lax.iota(jnp.int32, _LANES) + cg * _LANES,
                        jnp.int32(63))
        for cg in range(_P // _LANES)
    ]
    sv = [
        lax.shift_right_logical(lax.iota(jnp.int32, _LANES) + cg * _LANES,
                                6)
        for cg in range(_P // _LANES)
    ]

    nt = (_NWIN + _NW - 1) // _NW

    def win(t):
        return wid + t * _NW

    def start_read(t, b):
        pltpu.async_copy(tT_hbm.at[:, pl.ds(win(t) * _P, _P)], fbufs[b],
                         srs[b])

    def wait_read(t, b):
        pltpu.make_async_copy(tT_hbm.at[:, pl.ds(win(t) * _P, _P)],
                              fbufs[b], srs[b]).wait()

    def start_wb(t, b):
        pltpu.async_copy(gbufs[b], tpk_hbm.at[pl.ds(win(t) * 64, 64), :],
                         sws[b])

    def wait_wb(t, b):
        pltpu.make_async_copy(gbufs[b],
                              tpk_hbm.at[pl.ds(win(t) * 64, 64), :],
                              sws[b]).wait()

    @pl.when(wid == 0)
    def _():
        pltpu.async_copy(tail_hbm, tb, st)
        pltpu.make_async_copy(tail_hbm, tb, st).wait()
        pltpu.async_copy(tb, tpk_hbm.at[pl.ds(_TAILP, 32), :], st)
        pltpu.make_async_copy(tb, tpk_hbm.at[pl.ds(_TAILP, 32), :],
                              st).wait()

    @pl.when(win(0) < _NWIN)
    def _():
        start_read(0, 0)

    @pl.when(win(1) < _NWIN)
    def _():
        start_read(1, 1)

    def step(tt, carry):
        for b in range(2):
            t = tt * 2 + b

            @pl.when(win(t) < _NWIN)
            def _():
                @pl.when(t >= 2)
                def _():
                    wait_wb(t - 2, b)

                wait_read(t, b)
                for cg in range(_P // _LANES):
                    kvec = kv[cg]
                    soff = sv[cg]

                    @plsc.parallel_loop(0, 64, unroll=8)
                    def _(p):
                        vcol = soff + 2 * p
                        vals = plsc.load_gather(fbufs[b], [kvec, vcol])
                        gbufs[b][p, pl.ds(cg * _LANES, _LANES)] = vals

                @pl.when(win(t + 2) < _NWIN)
                def _():
                    start_read(t + 2, b)

                start_wb(t, b)
        return carry

    lax.fori_loop(0, (nt + 1) // 2, step, 0)

    @pl.when(win(nt - 2) < _NWIN)
    def _():
        wait_wb(nt - 2, (nt - 2) % 2)

    @pl.when(win(nt - 1) < _NWIN)
    def _():
        wait_wb(nt - 1, (nt - 1) % 2)


def _emb_kernel(tpk_hbm, xT_hbm, out_hbm, idx_v,
                pv0, pv1, pv2, pv3, rb0, rb1, rb2, rb3,
                ob0, ob1, sg0, sg1, sg2, sg3, sw0, sw1):
    pbufs = (pv0, pv1, pv2, pv3)
    rbufs = (rb0, rb1, rb2, rb3)
    obufs = (ob0, ob1)
    sgs = (sg0, sg1, sg2, sg3)
    sws = (sw0, sw1)
    wid = lax.axis_index("s") * _NC + lax.axis_index("c")
    i0 = wid * _P
    pltpu.sync_copy(xT_hbm.at[:, pl.ds(i0, _P)], idx_v)

    dipre = [
        lax.iota(jnp.int32, _LANES) + dg * _LANES
        for dg in range(_P // _LANES)
    ]

    def fill_and_gather(j, bg):
        for dg in range(_P // _LANES):
            sl = pl.ds(dg * _LANES, _LANES)
            pbufs[bg][sl] = lax.shift_right_logical(idx_v[j, sl], 1)
        pltpu.async_copy(tpk_hbm.at[pbufs[bg]], rbufs[bg], sgs[bg])

    def wait_gather(bg):
        pltpu.make_async_copy(tpk_hbm.at[pbufs[bg]], rbufs[bg],
                              sgs[bg]).wait()

    def start_wb(j, bo):
        pltpu.async_copy(obufs[bo], out_hbm.at[j, :, pl.ds(i0, _P)],
                         sws[bo])

    def wait_wb(j, bo):
        pltpu.make_async_copy(obufs[bo], out_hbm.at[j, :, pl.ds(i0, _P)],
                              sws[bo]).wait()

    for u in range(_NG):
        fill_and_gather(u, u)

    def step(jj, carry):
        for u in range(_NG):
            j = jj * _NG + u
            bo = u % _NO

            @pl.when(j >= _NO)
            def _():
                wait_wb(j - _NO, bo)

            wait_gather(u)

            for dg in range(_P // _LANES):
                sl = pl.ds(dg * _LANES, _LANES)
                half = lax.shift_left(
                    lax.bitwise_and(idx_v[j, sl], jnp.int32(1)), 6)
                rows = dipre[dg]

                @plsc.parallel_loop(0, D_MODEL_K, unroll=8)
                def _(k):
                    col = half + k
                    vals = plsc.load_gather(rbufs[u], [rows, col])
                    obufs[bo][k, sl] = vals * SCALE_K

            @pl.when(j + _NG < _J)
            def _():
                fill_and_gather(j + _NG, u)

            start_wb(j, bo)
        return carry

    lax.fori_loop(0, _J // _NG, step, 0)
    wait_wb(_J - 2, 0)
    wait_wb(_J - 1, 1)


@jax.jit
def _run_all(tT, tail32, xT):
    mesh = plsc.VectorSubcoreMesh(core_axis_name="c", subcore_axis_name="s")
    cp = pltpu.CompilerParams(
        use_tc_tiling_on_sc=True, needs_layout_passes=False)
    fmt = functools.partial(
        pl.kernel,
        mesh=mesh,
        out_type=jax.ShapeDtypeStruct((_NPAIR, _P), jnp.float32),
        compiler_params=cp,
        scratch_types=(
            [pltpu.VMEM((D_MODEL_K, _P), jnp.float32)] * 2
            + [pltpu.VMEM((D_MODEL_K, _P), jnp.float32)] * 2
            + [pltpu.VMEM((32, _P), jnp.float32)]
            + [pltpu.SemaphoreType.DMA] * 5
        ),
    )(_fmt_kernel)
    tpk = fmt(tT, tail32)
    emb = functools.partial(
        pl.kernel,
        mesh=mesh,
        out_type=jax.ShapeDtypeStruct((_J, D_MODEL_K, _I), jnp.float32),
        compiler_params=cp,
        scratch_types=(
            [pltpu.VMEM((_J, _P), jnp.int32)]
            + [pltpu.VMEM((_P,), jnp.int32)] * _NG
            + [pltpu.VMEM((_P, _P), jnp.float32)] * _NG
            + [pltpu.VMEM((D_MODEL_K, _P), jnp.float32)] * _NO
            + [pltpu.SemaphoreType.DMA] * (_NG + _NO)
        ),
    )(_emb_kernel)
    return emb(tpk, xT)


def kernel(x, table):
    xT = jnp.transpose(x).astype(jnp.int32)
    tT = jnp.transpose(table)
    tail32 = lax.slice(
        table, (_NWIN * 128, 0), (_V, D_MODEL_K)).reshape(32, _P)
    outT = _run_all(tT, tail32, xT)
    return jnp.transpose(outT, (2, 0, 1))


# speedup vs baseline: 1.2362x; 1.2362x over previous
import functools

import jax
import jax.numpy as jnp
from jax import lax
from jax.experimental import pallas as pl
from jax.experimental.pallas import tpu as pltpu
from jax.experimental.pallas import tpu_sc as plsc

D_MODEL_K = 64
SCALE_K = 8.0

_NC = 2
_NS = 16
_NW = _NC * _NS
_LANES = 16

_B_TOTAL = 4096 * 200
_B_PER_W = _B_TOTAL // _NW
_CHUNK = 400
_N_CHUNKS = _B_PER_W // _CHUNK
_NBUF = 4


def _emb_kernel(table_hbm, idx_hbm, out_hbm, idx_v,
                rb0, rb1, rb2, rb3, sg0, sg1, sg2, sg3,
                sw0, sw1, sw2, sw3):
    bufs = (rb0, rb1, rb2, rb3)
    sgs = (sg0, sg1, sg2, sg3)
    sws = (sw0, sw1, sw2, sw3)
    wid = lax.axis_index("s") * _NC + lax.axis_index("c")
    wbase = wid * _B_PER_W
    pltpu.sync_copy(idx_hbm.at[pl.ds(wbase, _B_PER_W)], idx_v)

    def start_gather(g, b):
        pltpu.async_copy(
            table_hbm.at[idx_v.at[pl.ds(g * _CHUNK, _CHUNK)]], bufs[b], sgs[b])

    def wait_gather(g, b):
        pltpu.make_async_copy(
            table_hbm.at[idx_v.at[pl.ds(g * _CHUNK, _CHUNK)]], bufs[b],
            sgs[b]).wait()

    def start_wb(g, b):
        pltpu.async_copy(
            bufs[b], out_hbm.at[pl.ds(wbase + g * _CHUNK, _CHUNK)], sws[b])

    def wait_wb(g, b):
        pltpu.make_async_copy(
            bufs[b], out_hbm.at[pl.ds(wbase + g * _CHUNK, _CHUNK)],
            sws[b]).wait()

    start_gather(0, 0)
    start_gather(1, 1)

    def quad(q, carry):
        for b in range(_NBUF):
            g = q * _NBUF + b
            bn = (b + 2) % _NBUF

            @pl.when(g >= 2)
            def _():
                wait_wb(g - 2, bn)

            @pl.when(g + 2 < _N_CHUNKS)
            def _():
                start_gather(g + 2, bn)

            wait_gather(g, b)

            @plsc.parallel_loop(0, _CHUNK, unroll=4)
            def _(i):
                for j in range(D_MODEL_K // _LANES):
                    sl = pl.ds(j * _LANES, _LANES)
                    bufs[b][i, sl] = bufs[b][i, sl] * SCALE_K

            start_wb(g, b)
        return carry

    lax.fori_loop(0, _N_CHUNKS // _NBUF, quad, 0)
    wait_wb(_N_CHUNKS - 2, (_N_CHUNKS - 2) % _NBUF)
    wait_wb(_N_CHUNKS - 1, (_N_CHUNKS - 1) % _NBUF)


@jax.jit
def _emb_call(idx_flat, table):
    mesh = plsc.VectorSubcoreMesh(core_axis_name="c", subcore_axis_name="s")
    run = functools.partial(
        pl.kernel,
        mesh=mesh,
        out_type=jax.ShapeDtypeStruct((_B_TOTAL, D_MODEL_K), jnp.float32),
        compiler_params=pltpu.CompilerParams(use_tc_tiling_on_sc=False),
        scratch_types=(
            [pltpu.VMEM((_B_PER_W,), jnp.int32)]
            + [pltpu.VMEM((_CHUNK, D_MODEL_K), jnp.float32)] * _NBUF
            + [pltpu.SemaphoreType.DMA] * (2 * _NBUF)
        ),
    )(_emb_kernel)
    return run(table, idx_flat)


def kernel(x, table):
    idx_flat = x.reshape(-1).astype(jnp.int32)
    out = _emb_call(idx_flat, table)
    return out.reshape(x.shape + (D_MODEL_K,))
